# Initial kernel scaffold; baseline (speedup 1.0000x reference)
import jax
import jax.numpy as jnp
from jax.experimental import pallas as pl


def kernel(inputs, prev_c, prev_h, segment_ids, W_combined, b_combined, W_f, U_f, b_f):
    raise NotImplementedError("write your pallas kernel here")


# speedup vs baseline: 2.4033x; 2.4033x over previous
import functools

import jax
import jax.numpy as jnp
from jax import lax
from jax.experimental import pallas as pl
from jax.experimental.pallas import tpu as pltpu
from jax.experimental.pallas import tpu_sc as plsc

_NP = 10000
_NCH = 320000
_ED = 128
_HD = 128

_NC = 2
_NS = 16
_L = 16
_NW = _NC * _NS
_CPW = _NCH // _NW
_CHG = 400
_CHS = 200

_mesh = plsc.VectorSubcoreMesh(core_axis_name="c", subcore_axis_name="s")


def _sc_gather(fx, seg):
    @functools.partial(
        pl.kernel,
        out_type=jax.ShapeDtypeStruct((_NCH, _HD), jnp.float32),
        mesh=_mesh,
        scratch_types=[
            pltpu.VMEM((_CHG,), jnp.int32),
            pltpu.VMEM((_CHG, _HD), jnp.float32),
            pltpu.SemaphoreType.DMA,
        ],
    )
    def k(fx_hbm, seg_hbm, out_hbm, idx_v, rows_v, sem):
        cid = lax.axis_index("c")
        sid = lax.axis_index("s")
        base = (cid * _NS + sid) * _CPW

        @pl.loop(0, _CPW, step=_CHG)
        def _(off):
            pltpu.sync_copy(seg_hbm.at[pl.ds(base + off, _CHG)], idx_v)
            pltpu.async_copy(fx_hbm.at[idx_v], rows_v, sem).wait()
            pltpu.sync_copy(rows_v, out_hbm.at[pl.ds(base + off, _CHG)])

    return k(fx, seg)


def _sc_segsum(vals, seg):
    @functools.partial(
        pl.kernel,
        out_type=jax.ShapeDtypeStruct((_NC, _NP, _HD), jnp.float32),
        mesh=_mesh,
        scratch_types=[
            pltpu.VMEM((_CHS,), jnp.int32),
            pltpu.VMEM((_CHS, _HD), jnp.float32),
            pltpu.VMEM_SHARED((_NP, _HD), jnp.float32),
            pltpu.SemaphoreType.DMA,
        ],
    )
    def k(vals_hbm, seg_hbm, out_hbm, idx_v, rows_v, acc_sh, sem):
        cid = lax.axis_index("c")
        sid = lax.axis_index("s")
        base = (cid * _NS + sid) * _CPW

        @pl.loop(0, _CHS)
        def _(r):
            @pl.loop(0, _HD, step=_L)
            def _(col):
                rows_v[r, pl.ds(col, _L)] = jnp.zeros((_L,), jnp.float32)

        @pl.loop(sid * _CHS, _NP, step=_CHS * _NS)
        def _(r0):
            pltpu.sync_copy(rows_v, acc_sh.at[pl.ds(r0, _CHS)])

        plsc.subcore_barrier()

        @pl.loop(0, _CPW, step=_CHS)
        def _(off):
            pltpu.sync_copy(seg_hbm.at[pl.ds(base + off, _CHS)], idx_v)
            pltpu.sync_copy(vals_hbm.at[pl.ds(base + off, _CHS)], rows_v)
            pltpu.sync_copy(rows_v, acc_sh.at[idx_v], add=True)

        plsc.subcore_barrier()

        @pl.loop(sid * _CHS, _NP, step=_CHS * _NS)
        def _(r0):
            pltpu.sync_copy(acc_sh.at[pl.ds(r0, _CHS)], rows_v)
            pltpu.sync_copy(rows_v, out_hbm.at[cid, pl.ds(r0, _CHS)])

    return k(vals, seg)


def _tc_fx(inputs, W_f, b_f):
    def body(x_ref, w_ref, b_ref, o_ref):
        o_ref[...] = (
            jnp.dot(x_ref[...], w_ref[...], preferred_element_type=jnp.float32)
            + b_ref[...]
        )

    return pl.pallas_call(
        body,
        out_shape=jax.ShapeDtypeStruct((_NP, _HD), jnp.float32),
    )(inputs, W_f, b_f)


_MID_R = 2000


def _tc_mid(prev_h, F, prev_c, U_f):
    def body(h_ref, f_ref, c_ref, u_ref, o_ref):
        fh = jnp.dot(h_ref[...], u_ref[...], preferred_element_type=jnp.float32)
        o_ref[...] = jax.nn.sigmoid(fh + f_ref[...]) * c_ref[...]

    blk = pl.BlockSpec((_MID_R, _HD), lambda i: (i, 0))
    return pl.pallas_call(
        body,
        grid=(_NCH // _MID_R,),
        in_specs=[blk, blk, blk, pl.BlockSpec((_HD, _HD), lambda i: (0, 0))],
        out_specs=blk,
        out_shape=jax.ShapeDtypeStruct((_NCH, _HD), jnp.float32),
    )(prev_h, F, prev_c, U_f)


_FIN_R = 2000


def _tc_final(inputs, hpart, fpart, W_combined, b_combined):
    def body(x_ref, hp_ref, fp_ref, wc_ref, b_ref, oc_ref, oh_ref):
        ht = hp_ref[0] + hp_ref[1]
        fc_term = fp_ref[0] + fp_ref[1]
        z = (
            jnp.dot(x_ref[...], wc_ref[: _ED], preferred_element_type=jnp.float32)
            + jnp.dot(ht, wc_ref[_ED:], preferred_element_type=jnp.float32)
            + b_ref[...]
        )
        z_i = z[:, :_HD]
        z_o = z[:, _HD : 2 * _HD]
        z_u = z[:, 2 * _HD :]
        c = jax.nn.sigmoid(z_i) * jnp.tanh(z_u) + fc_term
        oc_ref[...] = c
        oh_ref[...] = jax.nn.sigmoid(z_o) * jnp.tanh(c)

    blk = pl.BlockSpec((_FIN_R, _HD), lambda i: (i, 0))
    pblk = pl.BlockSpec((_NC, _FIN_R, _HD), lambda i: (0, i, 0))
    return pl.pallas_call(
        body,
        grid=(_NP // _FIN_R,),
        in_specs=[
            blk,
            pblk,
            pblk,
            pl.BlockSpec((_ED + _HD, 3 * _HD), lambda i: (0, 0)),
            pl.BlockSpec((1, 3 * _HD), lambda i: (0, 0)),
        ],
        out_specs=[blk, blk],
        out_shape=[
            jax.ShapeDtypeStruct((_NP, _HD), jnp.float32),
            jax.ShapeDtypeStruct((_NP, _HD), jnp.float32),
        ],
    )(inputs, hpart, fpart, W_combined, b_combined)


def kernel(inputs, prev_c, prev_h, segment_ids, W_combined, b_combined, W_f, U_f, b_f):
    seg = segment_ids.astype(jnp.int32)
    fx = _tc_fx(inputs, W_f, b_f)
    hpart = _sc_segsum(prev_h, seg)
    F = _sc_gather(fx, seg)
    fc_mul = _tc_mid(prev_h, F, prev_c, U_f)
    fpart = _sc_segsum(fc_mul, seg)
    c, h = _tc_final(inputs, hpart, fpart, W_combined, b_combined)
    return (c, h)
